# baseline (speedup 1.0000x reference)
import functools

import jax
import jax.numpy as jnp
from jax import lax
from jax.experimental import pallas as pl
from jax.experimental.pallas import tpu as pltpu
from jax.experimental.pallas import tpu_sc as plsc

N, T = 4096, 200
D = 64
V = 1000000
B = N * T
NC, NS = 2, 16
NW = NC * NS
K = 128
NBLK = B // K
BLK_PER_W = NBLK // NW
NBUF = 4
NGRP = BLK_PER_W // NBUF
NB_N = N // K


@functools.partial(
    pl.kernel,
    mesh=plsc.VectorSubcoreMesh(core_axis_name="c", subcore_axis_name="s"),
    out_type=jax.ShapeDtypeStruct((T, D, N), jnp.float32),
    compiler_params=pltpu.CompilerParams(
        use_tc_tiling_on_sc=True, needs_layout_passes=False),
    scratch_types=(
        [pltpu.VMEM((BLK_PER_W, K), jnp.int32)]
        + [pltpu.VMEM((K, 128), jnp.float32)] * NBUF
        + [pltpu.VMEM((D, K), jnp.float32)] * NBUF
        + [pltpu.SemaphoreType.DMA] * (2 * NBUF)
    ),
)
def _gather_kernel(table_hbm, idx_hbm, out_hbm, idx_v, *scratch):
    rows = scratch[:NBUF]
    tbuf = scratch[NBUF:2 * NBUF]
    gsem = scratch[2 * NBUF:3 * NBUF]
    wsem = scratch[3 * NBUF:]
    wid = lax.axis_index("s") * NC + lax.axis_index("c")
    pltpu.sync_copy(idx_hbm.at[pl.ds(wid * BLK_PER_W, BLK_PER_W)], idx_v)
    base = wid * BLK_PER_W

    lane = lax.iota(jnp.int32, 16)
    row_sel = [lane + 16 * k for k in range(K // 16)]

    def start_gather(g, b):
        pltpu.async_copy(table_hbm.at[idx_v.at[g]], rows[b], gsem[b])

    def wait_gather(g, b):
        pltpu.make_async_copy(table_hbm.at[idx_v.at[g]], rows[b], gsem[b]).wait()

    def _dst(g):
        r = base + g
        return out_hbm.at[r >> 5, :, pl.ds((r & 31) * K, K)]

    def start_wb(g, b):
        pltpu.async_copy(tbuf[b], _dst(g), wsem[b])

    def wait_wb(g, b):
        pltpu.make_async_copy(tbuf[b], _dst(g), wsem[b]).wait()

    def transpose(b):
        def per_d8(i, carry):
            d0 = i * 8
            for dd in range(8):
                col = lax.broadcast(d0 + dd, (16,))
                for k in range(K // 16):
                    vals = plsc.load_gather(rows[b], [row_sel[k], col])
                    tbuf[b][d0 + dd, pl.ds(16 * k, 16)] = vals
            return carry

        lax.fori_loop(0, D // 8, per_d8, 0)

    for b in range(NBUF):
        start_gather(b, b)

    for b in range(NBUF):
        wait_gather(b, b)
        transpose(b)
        start_wb(b, b)
        start_gather(NBUF + b, b)

    def group(i, carry):
        g0 = i * NBUF
        for b in range(NBUF):
            g = g0 + b
            wait_gather(g, b)
            wait_wb(g - NBUF, b)
            transpose(b)
            start_wb(g, b)
            start_gather(g + NBUF, b)
        return carry

    lax.fori_loop(1, NGRP - 1, group, 0)

    g0 = (NGRP - 1) * NBUF
    for b in range(NBUF):
        g = g0 + b
        wait_gather(g, b)
        wait_wb(g - NBUF, b)
        transpose(b)
        start_wb(g, b)
    for b in range(NBUF):
        wait_wb(g0 + b, b)


def kernel(x, W_embed):
    idx = jnp.transpose(x).reshape(NBLK, K).astype(jnp.int32)
    Wp = jnp.pad(W_embed, ((0, 0), (0, 128 - D)))
    out = _gather_kernel(Wp, idx)
    return out.transpose(2, 0, 1)


# speedup vs baseline: 1.4322x; 1.4322x over previous
import functools

import jax
import jax.numpy as jnp
from jax import lax
from jax.experimental import pallas as pl
from jax.experimental.pallas import tpu as pltpu
from jax.experimental.pallas import tpu_sc as plsc

N, T = 4096, 200
D = 64
B = N * T
NC, NS = 2, 16
NW = NC * NS
K = 128
ROWS_PER_W = B // NW
STEPS = ROWS_PER_W // K
NBUF = 4
NGRP = STEPS // NBUF


@functools.partial(
    pl.kernel,
    mesh=plsc.VectorSubcoreMesh(core_axis_name="c", subcore_axis_name="s"),
    out_type=jax.ShapeDtypeStruct((B, D), jnp.float32),
    compiler_params=pltpu.CompilerParams(use_tc_tiling_on_sc=False),
    scratch_types=(
        [pltpu.VMEM((STEPS, K), jnp.int32)]
        + [pltpu.VMEM((K, D), jnp.float32)] * NBUF
        + [pltpu.SemaphoreType.DMA] * (2 * NBUF)
    ),
)
def _gather_kernel(table_hbm, idx_hbm, out_hbm, idx_v, *scratch):
    rows = scratch[:NBUF]
    gsem = scratch[NBUF:2 * NBUF]
    wsem = scratch[2 * NBUF:]
    wid = lax.axis_index("s") * NC + lax.axis_index("c")
    pltpu.sync_copy(idx_hbm.at[pl.ds(wid * STEPS, STEPS)], idx_v)
    base = wid * ROWS_PER_W

    def start_gather(g, b):
        pltpu.async_copy(table_hbm.at[idx_v.at[g]], rows[b], gsem[b])

    def wait_gather(g, b):
        pltpu.make_async_copy(table_hbm.at[idx_v.at[g]], rows[b], gsem[b]).wait()

    def start_wb(g, b):
        pltpu.async_copy(rows[b], out_hbm.at[pl.ds(base + g * K, K)], wsem[b])

    def wait_wb(g, b):
        pltpu.make_async_copy(rows[b], out_hbm.at[pl.ds(base + g * K, K)], wsem[b]).wait()

    for b in range(NBUF):
        start_gather(b, b)

    def group(i, carry):
        g0 = i * NBUF
        for b in range(NBUF):
            wait_gather(g0 + b, b)
            start_wb(g0 + b, b)
        for b in range(NBUF):
            wait_wb(g0 + b, b)
            start_gather(g0 + NBUF + b, b)
        return carry

    lax.fori_loop(0, NGRP - 1, group, 0)

    g0 = (NGRP - 1) * NBUF
    for b in range(NBUF):
        wait_gather(g0 + b, b)
        start_wb(g0 + b, b)
    for b in range(NBUF):
        wait_wb(g0 + b, b)


def kernel(x, W_embed):
    idx = x.reshape(B // K, K).astype(jnp.int32)
    out = _gather_kernel(W_embed, idx)
    return out.reshape(N, T, D)


# speedup vs baseline: 1.4919x; 1.0417x over previous
import functools

import jax
import jax.numpy as jnp
from jax import lax
from jax.experimental import pallas as pl
from jax.experimental.pallas import tpu as pltpu
from jax.experimental.pallas import tpu_sc as plsc

N, T = 4096, 200
D = 64
V = 1000000
B = N * T
NC, NS = 2, 16
NW = NC * NS
K = 128
NBLK = B // K
BLK_PER_W = NBLK // NW
NBUF = 4
NGRP = BLK_PER_W // NBUF
NB_N = N // K


@functools.partial(
    pl.kernel,
    mesh=plsc.VectorSubcoreMesh(core_axis_name="c", subcore_axis_name="s"),
    out_type=jax.ShapeDtypeStruct((T, D, N), jnp.float32),
    compiler_params=pltpu.CompilerParams(
        use_tc_tiling_on_sc=True, needs_layout_passes=False),
    scratch_types=(
        [pltpu.VMEM((BLK_PER_W, K), jnp.int32)]
        + [pltpu.VMEM((K, 128), jnp.float32)] * NBUF
        + [pltpu.VMEM((D, K), jnp.float32)] * NBUF
        + [pltpu.SemaphoreType.DMA] * (2 * NBUF)
    ),
)
def _gather_kernel(table_hbm, idx_hbm, out_hbm, idx_v, *scratch):
    rows = scratch[:NBUF]
    tbuf = scratch[NBUF:2 * NBUF]
    gsem = scratch[2 * NBUF:3 * NBUF]
    wsem = scratch[3 * NBUF:]
    wid = lax.axis_index("s") * NC + lax.axis_index("c")
    pltpu.sync_copy(idx_hbm.at[pl.ds(wid * BLK_PER_W, BLK_PER_W)], idx_v)
    base = wid * BLK_PER_W

    lane = lax.iota(jnp.int32, 16)
    row_sel = [lane + 16 * k for k in range(K // 16)]

    def start_gather(g, b):
        pltpu.async_copy(table_hbm.at[idx_v.at[g]], rows[b], gsem[b])

    def wait_gather(g, b):
        pltpu.make_async_copy(table_hbm.at[idx_v.at[g]], rows[b], gsem[b]).wait()

    def _dst(g):
        r = base + g
        return out_hbm.at[r >> 5, :, pl.ds((r & 31) * K, K)]

    def start_wb(g, b):
        pltpu.async_copy(tbuf[b], _dst(g), wsem[b])

    def wait_wb(g, b):
        pltpu.make_async_copy(tbuf[b], _dst(g), wsem[b]).wait()

    def transpose(b):
        @plsc.parallel_loop(0, D, unroll=8)
        def _per_d(d):
            col = lax.broadcast(d, (16,))
            for k in range(K // 16):
                vals = plsc.load_gather(rows[b], [row_sel[k], col])
                tbuf[b][d, pl.ds(16 * k, 16)] = vals

    for b in range(NBUF):
        start_gather(b, b)

    for b in range(NBUF):
        wait_gather(b, b)
        transpose(b)
        start_wb(b, b)
        start_gather(NBUF + b, b)

    def group(i, carry):
        g0 = i * NBUF
        for b in range(NBUF):
            g = g0 + b
            wait_gather(g, b)
            wait_wb(g - NBUF, b)
            transpose(b)
            start_wb(g, b)
            start_gather(g + NBUF, b)
        return carry

    lax.fori_loop(1, NGRP - 1, group, 0)

    g0 = (NGRP - 1) * NBUF
    for b in range(NBUF):
        g = g0 + b
        wait_gather(g, b)
        wait_wb(g - NBUF, b)
        transpose(b)
        start_wb(g, b)
    for b in range(NBUF):
        wait_wb(g0 + b, b)


def kernel(x, W_embed):
    idx = jnp.transpose(x).reshape(NBLK, K).astype(jnp.int32)
    Wp = jnp.pad(W_embed, ((0, 0), (0, 128 - D)))
    out = _gather_kernel(Wp, idx)
    return out.transpose(2, 0, 1)
